# baseline (speedup 1.0000x reference)
import functools

import jax
import jax.numpy as jnp
from jax import lax
from jax.experimental import pallas as pl
from jax.experimental.pallas import tpu as pltpu
from jax.experimental.pallas import tpu_sc as plsc

E = 64
B = 16384
L = 12
TOK = B * L
NC, NS = 2, 16
NW = NC * NS
C = 128
N_CHUNKS = L * (B // NW) // C


def _sc_gather_sum(ts2, ti2, ta2, tact2, tab_s, tab_i, tab_a, tab_act):
    mesh = plsc.VectorSubcoreMesh(core_axis_name="c", subcore_axis_name="s")

    @functools.partial(
        pl.kernel,
        mesh=mesh,
        out_type=jax.ShapeDtypeStruct((TOK // 2, 2 * E), jnp.float32),
        compiler_params=pltpu.CompilerParams(use_tc_tiling_on_sc=False),
        scratch_types=[
            pltpu.VMEM((L, B // NW), jnp.int32),
            pltpu.VMEM((L, B // NW), jnp.int32),
            pltpu.VMEM((L, B // NW), jnp.int32),
            pltpu.VMEM((L, B // NW), jnp.int32),
            pltpu.VMEM((C, E), jnp.float32),
            pltpu.VMEM((C, E), jnp.float32),
            pltpu.VMEM((C, E), jnp.float32),
            pltpu.VMEM((C, E), jnp.float32),
            pltpu.VMEM((C, E), jnp.float32),
            pltpu.VMEM((C, E), jnp.float32),
            pltpu.VMEM((C, E), jnp.float32),
            pltpu.VMEM((C, E), jnp.float32),
            pltpu.VMEM((C, E), jnp.float32),
            pltpu.VMEM((C, E), jnp.float32),
            pltpu.SemaphoreType.DMA,
            pltpu.SemaphoreType.DMA,
            pltpu.SemaphoreType.DMA,
            pltpu.SemaphoreType.DMA,
        ],
    )
    def k(ts_h, ti_h, ta_h, tact_h, tabs_h, tabi_h, taba_h, tabact_h, out_h,
          s0, s1, s2, s3,
          b00, b01, b02, b03, b10, b11, b12, b13,
          o0, o1,
          sg0, sg1, so0, so1):
        wid = lax.axis_index("s") * NC + lax.axis_index("c")
        wb0 = wid * (B // NW)

        slabs = (s0, s1, s2, s3)
        toks = (ts_h, ti_h, ta_h, tact_h)
        tabs = (tabs_h, tabi_h, taba_h, tabact_h)
        bufs = ((b00, b01, b02, b03), (b10, b11, b12, b13))
        obufs = (o0, o1)
        gsems = (sg0, sg1)
        osems = (so0, so1)

        for t in range(4):
            for l in range(L):
                pltpu.async_copy(toks[t].at[l, pl.ds(wb0, B // NW)],
                                 slabs[t].at[l], sg0)
        for t in range(4):
            for l in range(L):
                pltpu.make_async_copy(toks[t].at[l, pl.ds(0, B // NW)],
                                      slabs[t].at[l], sg0).wait()

        def fire(g, slot):
            l = lax.div(g, 4)
            boff = lax.rem(g, 4) * C
            for t in range(4):
                idx = slabs[t].at[l, pl.ds(boff, C)]
                pltpu.async_copy(tabs[t].at[idx], bufs[slot][t],
                                 gsems[slot])

        def wait_gathers(slot):
            for t in range(4):
                pltpu.make_async_copy(tabs[t].at[pl.ds(0, C)],
                                      bufs[slot][t], gsems[slot]).wait()

        def wait_out(slot):
            pltpu.make_async_copy(
                obufs[slot], out_h.at[pl.ds(0, C), pl.ds(0, E)],
                osems[slot]).wait()

        def do_sum(g, slot):
            l = lax.div(g, 4)
            boff = lax.rem(g, 4) * C
            bt = bufs[slot]
            ob = obufs[slot]

            def group(kk, carry):
                mv = [jnp.where(
                    slabs[t][l, pl.ds(boff + kk * 16, 16)] == 0, 0.0, 1.0)
                    for t in range(4)]
                for j in range(16):
                    r = 16 * kk + j
                    for q in range(4):
                        sl = pl.ds(q * 16, 16)
                        ob[r, sl] = (
                            mv[0][j] * bt[0][r, sl] + mv[1][j] * bt[1][r, sl]
                            + mv[2][j] * bt[2][r, sl]
                            + mv[3][j] * bt[3][r, sl])
                return carry

            lax.fori_loop(0, C // 16, group, 0)

        fire(0, 0)

        def chunk_pair(gg, carry):
            for slot in range(2):
                g = 2 * gg + slot

                @pl.when(g + 1 < N_CHUNKS)
                def _():
                    fire(g + 1, 1 - slot)

                wait_gathers(slot)

                @pl.when(g >= 2)
                def _():
                    wait_out(slot)

                do_sum(g, slot)
                l = lax.div(g, 4)
                boff = lax.rem(g, 4) * C
                prow = lax.div(l, 2) * B + wb0 + boff
                hoff = lax.rem(l, 2) * E
                pltpu.async_copy(
                    obufs[slot],
                    out_h.at[pl.ds(prow, C), pl.ds(hoff, E)],
                    osems[slot])
            return carry

        lax.fori_loop(0, N_CHUNKS // 2, chunk_pair, 0)
        wait_out(0)
        wait_out(1)

    return k(ts2, ti2, ta2, tact2, tab_s, tab_i, tab_a, tab_act)


NBB = 512
N_TCB = B // NBB


def _tc_body(x0, x1, x2, x3, x4, x5, w_ref, o_ref):
    wv = w_ref[...]
    for j, xr in enumerate((x0, x1, x2, x3, x4, x5)):
        xj = xr[...]
        for h in range(2):
            xl = xj[:, h * E:(h + 1) * E]
            y = lax.dot_general(wv, xl, (((0,), (1,)), ((), ())))
            o_ref[2 * j + h] = jnp.maximum(y, 0.0)


def _tc_project(x, w):
    xspecs = [
        pl.BlockSpec((NBB, 2 * E), lambda i, j=j: (j * N_TCB + i, 0))
        for j in range(6)
    ]
    return pl.pallas_call(
        _tc_body,
        grid=(N_TCB,),
        in_specs=xspecs + [pl.BlockSpec((E, E), lambda i: (0, 0))],
        out_specs=pl.BlockSpec((L, E, NBB), lambda i: (0, 0, i)),
        out_shape=jax.ShapeDtypeStruct((L, E, B), jnp.float32),
    )(x, x, x, x, x, x, w)


def kernel(species_tokens, items_tokens, abilities_tokens, actions_tokens,
           species_table, items_table, abilities_table, actions_table,
           W_combine):
    tokst = [jnp.pad(t.astype(jnp.int32).T, ((0, 4), (0, 0))) for t in
             (species_tokens, items_tokens, abilities_tokens, actions_tokens)]

    combined = _sc_gather_sum(*tokst, species_table, items_table,
                              abilities_table, actions_table)

    out_t = _tc_project(combined, W_combine)
    return jnp.transpose(out_t, (2, 0, 1))
